# Initial kernel scaffold; baseline (speedup 1.0000x reference)
import jax
import jax.numpy as jnp
from jax.experimental import pallas as pl


def kernel(x, edge_index, W_self1, W_neigh1, b1, W_self2, W_neigh2, b2):
    raise NotImplementedError("write your pallas kernel here")


# speedup vs baseline: 5.2291x; 5.2291x over previous
import functools

import jax
import jax.numpy as jnp
from jax import lax
from jax.experimental import pallas as pl
from jax.experimental.pallas import tpu as pltpu
from jax.experimental.pallas import tpu_sc as plsc

N = 10000
E = 320000
D = 128
NC = 2
NS = 16
NW = NC * NS
EPW = E // NW
C = 128
FULL = EPW // C
TAIL = EPW - FULL * C
RPT = 624
REM = N - NS * RPT

_mesh = plsc.VectorSubcoreMesh(core_axis_name="c", subcore_axis_name="s")


def _agg_body(h, src, dst, zrow, msum_o,
              acc, sidx, didx, rows, sidx_t, didx_t, rows_t, sem):
    cid = lax.axis_index("c")
    sid = lax.axis_index("s")
    wid = cid * NS + sid
    pltpu.sync_copy(zrow, acc.at[pl.ds(sid * RPT, RPT)])

    @pl.when(sid == 0)
    def _():
        pltpu.sync_copy(zrow.at[pl.ds(0, REM)], acc.at[pl.ds(NS * RPT, REM)])

    plsc.subcore_barrier()
    base = wid * EPW

    @pl.loop(0, FULL)
    def _(i):
        off = base + i * C
        pltpu.sync_copy(src.at[pl.ds(off, C)], sidx)
        pltpu.sync_copy(dst.at[pl.ds(off, C)], didx)
        pltpu.async_copy(h.at[sidx], rows, sem).wait()
        pltpu.sync_copy(rows, acc.at[didx], add=True)

    off = base + FULL * C
    pltpu.sync_copy(src.at[pl.ds(off, TAIL)], sidx_t)
    pltpu.sync_copy(dst.at[pl.ds(off, TAIL)], didx_t)
    pltpu.async_copy(h.at[sidx_t], rows_t, sem).wait()
    pltpu.sync_copy(rows_t, acc.at[didx_t], add=True)

    plsc.subcore_barrier()
    pltpu.sync_copy(acc.at[pl.ds(sid * RPT, RPT)],
                    msum_o.at[cid, pl.ds(sid * RPT, RPT)])

    @pl.when(sid == 0)
    def _():
        pltpu.sync_copy(acc.at[pl.ds(NS * RPT, REM)],
                        msum_o.at[cid, pl.ds(NS * RPT, REM)])


def _deg_body(dst, zrow, ones_h, deg_o,
              acc, didx, ones_v, didx_t, sem):
    cid = lax.axis_index("c")
    sid = lax.axis_index("s")
    wid = cid * NS + sid
    pltpu.sync_copy(zrow, acc.at[pl.ds(sid * RPT, RPT)])

    @pl.when(sid == 0)
    def _():
        pltpu.sync_copy(zrow.at[pl.ds(0, REM)], acc.at[pl.ds(NS * RPT, REM)])

    pltpu.sync_copy(ones_h, ones_v)
    plsc.subcore_barrier()
    base = wid * EPW

    @pl.loop(0, FULL)
    def _(i):
        pltpu.sync_copy(dst.at[pl.ds(base + i * C, C)], didx)
        pltpu.sync_copy(ones_v, acc.at[didx], add=True)

    pltpu.sync_copy(dst.at[pl.ds(base + FULL * C, TAIL)], didx_t)
    pltpu.sync_copy(ones_v.at[pl.ds(0, TAIL)], acc.at[didx_t], add=True)

    plsc.subcore_barrier()
    pltpu.sync_copy(acc.at[pl.ds(sid * RPT, RPT)],
                    deg_o.at[cid, pl.ds(sid * RPT, RPT)])

    @pl.when(sid == 0)
    def _():
        pltpu.sync_copy(acc.at[pl.ds(NS * RPT, REM)],
                        deg_o.at[cid, pl.ds(NS * RPT, REM)])


_agg = pl.kernel(
    _agg_body,
    out_type=jax.ShapeDtypeStruct((NC, N, D), jnp.float32),
    mesh=_mesh,
    scratch_types=[
        pltpu.VMEM_SHARED((N, D), jnp.float32),
        pltpu.VMEM((C,), jnp.int32),
        pltpu.VMEM((C,), jnp.int32),
        pltpu.VMEM((C, D), jnp.float32),
        pltpu.VMEM((TAIL,), jnp.int32),
        pltpu.VMEM((TAIL,), jnp.int32),
        pltpu.VMEM((TAIL, D), jnp.float32),
        pltpu.SemaphoreType.DMA,
    ],
)

_deg = pl.kernel(
    _deg_body,
    out_type=jax.ShapeDtypeStruct((NC, N, D), jnp.float32),
    mesh=_mesh,
    scratch_types=[
        pltpu.VMEM_SHARED((N, D), jnp.float32),
        pltpu.VMEM((C,), jnp.int32),
        pltpu.VMEM((C, D), jnp.float32),
        pltpu.VMEM((TAIL,), jnp.int32),
        pltpu.SemaphoreType.DMA,
    ],
)

RB = 1000


def _tc_layer_body(x_ref, m_ref, d_ref, ws_ref, wn_ref, b_ref, o_ref, *,
                   relu):
    msum = m_ref[0] + m_ref[1]
    deg = jnp.maximum(d_ref[0, :, 0] + d_ref[1, :, 0], 1.0)
    agg = lax.dot(msum, wn_ref[...], precision=lax.Precision.HIGHEST)
    slf = lax.dot(x_ref[...], ws_ref[...], precision=lax.Precision.HIGHEST)
    out = slf + agg / deg[:, None] + b_ref[...]
    o_ref[...] = jnp.maximum(out, 0.0) if relu else out


def _make_tc_layer(relu):
    return pl.pallas_call(
        functools.partial(_tc_layer_body, relu=relu),
        grid=(N // RB,),
        in_specs=[
            pl.BlockSpec((RB, D), lambda i: (i, 0)),
            pl.BlockSpec((NC, RB, D), lambda i: (0, i, 0)),
            pl.BlockSpec((NC, RB, D), lambda i: (0, i, 0)),
            pl.BlockSpec((D, D), lambda i: (0, 0)),
            pl.BlockSpec((D, D), lambda i: (0, 0)),
            pl.BlockSpec((1, D), lambda i: (0, 0)),
        ],
        out_specs=pl.BlockSpec((RB, D), lambda i: (i, 0)),
        out_shape=jax.ShapeDtypeStruct((N, D), jnp.float32),
    )


_tc_layer1 = _make_tc_layer(relu=True)
_tc_layer2 = _make_tc_layer(relu=False)


def kernel(x, edge_index, W_self1, W_neigh1, b1, W_self2, W_neigh2, b2):
    src = edge_index[0]
    dst = edge_index[1]
    zrow = jnp.zeros((RPT, D), jnp.float32)
    ones_h = jnp.ones((C, D), jnp.float32)
    degp = _deg(dst, zrow, ones_h)
    msum1 = _agg(x, src, dst, zrow)
    h1 = _tc_layer1(x, msum1, degp, W_self1, W_neigh1, b1.reshape(1, D))
    msum2 = _agg(h1, src, dst, zrow)
    return _tc_layer2(h1, msum2, degp, W_self2, W_neigh2, b2.reshape(1, D))


# speedup vs baseline: 7.2845x; 1.3931x over previous
import functools

import jax
import jax.numpy as jnp
from jax import lax
from jax.experimental import pallas as pl
from jax.experimental.pallas import tpu as pltpu
from jax.experimental.pallas import tpu_sc as plsc

N = 10000
E = 320000
D = 128
NC = 2
NS = 16
NW = NC * NS
EPW = E // NW
C = 128
FULL = EPW // C
TAIL = EPW - FULL * C
RPT = 624
REM = N - NS * RPT

_mesh = plsc.VectorSubcoreMesh(core_axis_name="c", subcore_axis_name="s")


def _agg_body(h, src, dst, zrow, msum_o,
              acc, sidx0, didx0, rows0, sidx1, didx1, rows1,
              sidx_t, didx_t, rows_t, sem0, sem1):
    cid = lax.axis_index("c")
    sid = lax.axis_index("s")
    wid = cid * NS + sid
    pltpu.sync_copy(zrow, acc.at[pl.ds(sid * RPT, RPT)])

    @pl.when(sid == 0)
    def _():
        pltpu.sync_copy(zrow.at[pl.ds(0, REM)], acc.at[pl.ds(NS * RPT, REM)])

    plsc.subcore_barrier()
    base = wid * EPW

    def start(off, sidx, didx, rows, sem):
        pltpu.sync_copy(src.at[pl.ds(off, C)], sidx)
        pltpu.sync_copy(dst.at[pl.ds(off, C)], didx)
        pltpu.async_copy(h.at[sidx], rows, sem)

    start(base, sidx0, didx0, rows0, sem0)

    @pl.loop(0, FULL // 2)
    def _(j):
        c0 = base + (2 * j) * C
        start(c0 + C, sidx1, didx1, rows1, sem1)
        pltpu.make_async_copy(h.at[sidx0], rows0, sem0).wait()
        pltpu.sync_copy(rows0, acc.at[didx0], add=True)

        @pl.when(j < FULL // 2 - 1)
        def _():
            start(c0 + 2 * C, sidx0, didx0, rows0, sem0)

        pltpu.make_async_copy(h.at[sidx1], rows1, sem1).wait()
        pltpu.sync_copy(rows1, acc.at[didx1], add=True)

    off = base + FULL * C
    pltpu.sync_copy(src.at[pl.ds(off, TAIL)], sidx_t)
    pltpu.sync_copy(dst.at[pl.ds(off, TAIL)], didx_t)
    pltpu.async_copy(h.at[sidx_t], rows_t, sem0).wait()
    pltpu.sync_copy(rows_t, acc.at[didx_t], add=True)

    plsc.subcore_barrier()
    pltpu.sync_copy(acc.at[pl.ds(sid * RPT, RPT)],
                    msum_o.at[cid, pl.ds(sid * RPT, RPT)])

    @pl.when(sid == 0)
    def _():
        pltpu.sync_copy(acc.at[pl.ds(NS * RPT, REM)],
                        msum_o.at[cid, pl.ds(NS * RPT, REM)])


def _deg_body(dst, zrow, ones_h, deg_o,
              acc, didx, ones_v, didx_t, sem):
    cid = lax.axis_index("c")
    sid = lax.axis_index("s")
    wid = cid * NS + sid
    pltpu.sync_copy(zrow, acc.at[pl.ds(sid * RPT, RPT)])

    @pl.when(sid == 0)
    def _():
        pltpu.sync_copy(zrow.at[pl.ds(0, REM)], acc.at[pl.ds(NS * RPT, REM)])

    pltpu.sync_copy(ones_h, ones_v)
    plsc.subcore_barrier()
    base = wid * EPW

    @pl.loop(0, FULL)
    def _(i):
        pltpu.sync_copy(dst.at[pl.ds(base + i * C, C)], didx)
        pltpu.sync_copy(ones_v, acc.at[didx], add=True)

    pltpu.sync_copy(dst.at[pl.ds(base + FULL * C, TAIL)], didx_t)
    pltpu.sync_copy(ones_v.at[pl.ds(0, TAIL)], acc.at[didx_t], add=True)

    plsc.subcore_barrier()
    pltpu.sync_copy(acc.at[pl.ds(sid * RPT, RPT)],
                    deg_o.at[cid, pl.ds(sid * RPT, RPT)])

    @pl.when(sid == 0)
    def _():
        pltpu.sync_copy(acc.at[pl.ds(NS * RPT, REM)],
                        deg_o.at[cid, pl.ds(NS * RPT, REM)])


_agg = pl.kernel(
    _agg_body,
    out_type=jax.ShapeDtypeStruct((NC, N, D), jnp.float32),
    mesh=_mesh,
    scratch_types=[
        pltpu.VMEM_SHARED((N, D), jnp.float32),
        pltpu.VMEM((C,), jnp.int32),
        pltpu.VMEM((C,), jnp.int32),
        pltpu.VMEM((C, D), jnp.float32),
        pltpu.VMEM((C,), jnp.int32),
        pltpu.VMEM((C,), jnp.int32),
        pltpu.VMEM((C, D), jnp.float32),
        pltpu.VMEM((TAIL,), jnp.int32),
        pltpu.VMEM((TAIL,), jnp.int32),
        pltpu.VMEM((TAIL, D), jnp.float32),
        pltpu.SemaphoreType.DMA,
        pltpu.SemaphoreType.DMA,
    ],
)

_deg = pl.kernel(
    _deg_body,
    out_type=jax.ShapeDtypeStruct((NC, N, D), jnp.float32),
    mesh=_mesh,
    scratch_types=[
        pltpu.VMEM_SHARED((N, D), jnp.float32),
        pltpu.VMEM((C,), jnp.int32),
        pltpu.VMEM((C, D), jnp.float32),
        pltpu.VMEM((TAIL,), jnp.int32),
        pltpu.SemaphoreType.DMA,
    ],
)

RB = 1000


def _tc_layer_body(x_ref, m_ref, d_ref, ws_ref, wn_ref, b_ref, o_ref, *,
                   relu):
    msum = m_ref[0] + m_ref[1]
    deg = jnp.maximum(d_ref[0, :, 0] + d_ref[1, :, 0], 1.0)
    agg = lax.dot(msum, wn_ref[...], precision=lax.Precision.HIGHEST)
    slf = lax.dot(x_ref[...], ws_ref[...], precision=lax.Precision.HIGHEST)
    out = slf + agg / deg[:, None] + b_ref[...]
    o_ref[...] = jnp.maximum(out, 0.0) if relu else out


def _make_tc_layer(relu):
    return pl.pallas_call(
        functools.partial(_tc_layer_body, relu=relu),
        grid=(N // RB,),
        in_specs=[
            pl.BlockSpec((RB, D), lambda i: (i, 0)),
            pl.BlockSpec((NC, RB, D), lambda i: (0, i, 0)),
            pl.BlockSpec((NC, RB, D), lambda i: (0, i, 0)),
            pl.BlockSpec((D, D), lambda i: (0, 0)),
            pl.BlockSpec((D, D), lambda i: (0, 0)),
            pl.BlockSpec((1, D), lambda i: (0, 0)),
        ],
        out_specs=pl.BlockSpec((RB, D), lambda i: (i, 0)),
        out_shape=jax.ShapeDtypeStruct((N, D), jnp.float32),
    )


_tc_layer1 = _make_tc_layer(relu=True)
_tc_layer2 = _make_tc_layer(relu=False)


def kernel(x, edge_index, W_self1, W_neigh1, b1, W_self2, W_neigh2, b2):
    src = edge_index[0]
    dst = edge_index[1]
    zrow = jnp.zeros((RPT, D), jnp.float32)
    ones_h = jnp.ones((C, D), jnp.float32)
    degp = _deg(dst, zrow, ones_h)
    msum1 = _agg(x, src, dst, zrow)
    h1 = _tc_layer1(x, msum1, degp, W_self1, W_neigh1, b1.reshape(1, D))
    msum2 = _agg(h1, src, dst, zrow)
    return _tc_layer2(h1, msum2, degp, W_self2, W_neigh2, b2.reshape(1, D))


# speedup vs baseline: 7.6079x; 1.0444x over previous
import functools

import jax
import jax.numpy as jnp
from jax import lax
from jax.experimental import pallas as pl
from jax.experimental.pallas import tpu as pltpu
from jax.experimental.pallas import tpu_sc as plsc

N = 10000
E = 320000
D = 128
NC = 2
NS = 16
NW = NC * NS
EPW = E // NW
C = 128
FULL = EPW // C
TAIL = EPW - FULL * C
RPT = 624
REM = N - NS * RPT

_mesh = plsc.VectorSubcoreMesh(core_axis_name="c", subcore_axis_name="s")


def _agg_body(h, src, dst, zrow, msum_o,
              acc, sidx0, didx0, rows0, sidx1, didx1, rows1,
              sidx_t, didx_t, rows_t, sem0, sem1, semi0, semi1):
    _agg_pipeline(h, src, dst, zrow, msum_o, acc, sidx0, didx0, rows0,
                  sidx1, didx1, rows1, sidx_t, didx_t, rows_t,
                  sem0, sem1, semi0, semi1)


def _zero_acc(zrow, acc, sid):
    pltpu.sync_copy(zrow, acc.at[pl.ds(sid * RPT, RPT)])

    @pl.when(sid == 0)
    def _():
        pltpu.sync_copy(zrow.at[pl.ds(0, REM)], acc.at[pl.ds(NS * RPT, REM)])


def _drain_acc(acc, out, cid, sid):
    pltpu.sync_copy(acc.at[pl.ds(sid * RPT, RPT)],
                    out.at[cid, pl.ds(sid * RPT, RPT)])

    @pl.when(sid == 0)
    def _():
        pltpu.sync_copy(acc.at[pl.ds(NS * RPT, REM)],
                        out.at[cid, pl.ds(NS * RPT, REM)])


def _agg_pipeline(h, src, dst, zrow, msum_o,
                  acc, sidx0, didx0, rows0, sidx1, didx1, rows1,
                  sidx_t, didx_t, rows_t, sem0, sem1, semi0, semi1):
    cid = lax.axis_index("c")
    sid = lax.axis_index("s")
    wid = cid * NS + sid
    _zero_acc(zrow, acc, sid)
    plsc.subcore_barrier()
    base = wid * EPW
    _gather_scatter(h, src, dst, acc, sidx0, didx0, rows0, sidx1, didx1,
                    rows1, sem0, sem1, semi0, semi1, base, FULL)
    off = base + FULL * C
    pltpu.sync_copy(src.at[pl.ds(off, TAIL)], sidx_t)
    pltpu.sync_copy(dst.at[pl.ds(off, TAIL)], didx_t)
    pltpu.async_copy(h.at[sidx_t], rows_t, sem0).wait()
    pltpu.sync_copy(rows_t, acc.at[didx_t], add=True)
    plsc.subcore_barrier()
    _drain_acc(acc, msum_o, cid, sid)


def _gather_scatter(h, src, dst, acc, sidx0, didx0, rows0, sidx1, didx1,
                    rows1, sem0, sem1, semi0, semi1, base, n_full):
    def fire_idx(off, sidx, didx, semi):
        pltpu.async_copy(src.at[pl.ds(off, C)], sidx, semi)
        pltpu.async_copy(dst.at[pl.ds(off, C)], didx, semi)

    def wait_idx(sidx, didx, semi):
        pltpu.make_async_copy(src.at[pl.ds(base, C)], sidx, semi).wait()
        pltpu.make_async_copy(dst.at[pl.ds(base, C)], didx, semi).wait()

    fire_idx(base, sidx0, didx0, semi0)
    wait_idx(sidx0, didx0, semi0)
    pltpu.async_copy(h.at[sidx0], rows0, sem0)
    fire_idx(base + C, sidx1, didx1, semi1)
    LAST = n_full // 2 - 1

    @pl.loop(0, n_full // 2)
    def _(j):
        c0 = base + (2 * j) * C
        wait_idx(sidx1, didx1, semi1)
        pltpu.async_copy(h.at[sidx1], rows1, sem1)
        pltpu.make_async_copy(h.at[sidx0], rows0, sem0).wait()
        pltpu.sync_copy(rows0, acc.at[didx0], add=True)

        @pl.when(j < LAST)
        def _():
            fire_idx(c0 + 2 * C, sidx0, didx0, semi0)

        pltpu.make_async_copy(h.at[sidx1], rows1, sem1).wait()
        pltpu.sync_copy(rows1, acc.at[didx1], add=True)

        @pl.when(j < LAST)
        def _():
            fire_idx(c0 + 3 * C, sidx1, didx1, semi1)
            wait_idx(sidx0, didx0, semi0)
            pltpu.async_copy(h.at[sidx0], rows0, sem0)


def _make_agg(W):
    return pl.kernel(
        _agg_body,
        out_type=jax.ShapeDtypeStruct((NC, N, W), jnp.float32),
        mesh=_mesh,
        scratch_types=[
            pltpu.VMEM_SHARED((N, W), jnp.float32),
            pltpu.VMEM((C,), jnp.int32),
            pltpu.VMEM((C,), jnp.int32),
            pltpu.VMEM((C, W), jnp.float32),
            pltpu.VMEM((C,), jnp.int32),
            pltpu.VMEM((C,), jnp.int32),
            pltpu.VMEM((C, W), jnp.float32),
            pltpu.VMEM((TAIL,), jnp.int32),
            pltpu.VMEM((TAIL,), jnp.int32),
            pltpu.VMEM((TAIL, W), jnp.float32),
            pltpu.SemaphoreType.DMA,
            pltpu.SemaphoreType.DMA,
            pltpu.SemaphoreType.DMA,
            pltpu.SemaphoreType.DMA,
        ],
    )


_agg = _make_agg(D)


def _deg_body(dst, zrow, ones_h, deg_o,
              acc, didx, ones_v, didx_t, sem):
    cid = lax.axis_index("c")
    sid = lax.axis_index("s")
    wid = cid * NS + sid
    _zero_acc(zrow, acc, sid)
    pltpu.sync_copy(ones_h, ones_v)
    plsc.subcore_barrier()
    base = wid * EPW

    @pl.loop(0, FULL)
    def _(i):
        pltpu.sync_copy(dst.at[pl.ds(base + i * C, C)], didx)
        pltpu.sync_copy(ones_v, acc.at[didx], add=True)

    pltpu.sync_copy(dst.at[pl.ds(base + FULL * C, TAIL)], didx_t)
    pltpu.sync_copy(ones_v.at[pl.ds(0, TAIL)], acc.at[didx_t], add=True)

    plsc.subcore_barrier()
    _drain_acc(acc, deg_o, cid, sid)


_deg = pl.kernel(
    _deg_body,
    out_type=jax.ShapeDtypeStruct((NC, N, D), jnp.float32),
    mesh=_mesh,
    scratch_types=[
        pltpu.VMEM_SHARED((N, D), jnp.float32),
        pltpu.VMEM((C,), jnp.int32),
        pltpu.VMEM((C, D), jnp.float32),
        pltpu.VMEM((TAIL,), jnp.int32),
        pltpu.SemaphoreType.DMA,
    ],
)

RB = 1000
DEGW = 16


def _tc_layer_body(x_ref, m_ref, d_ref, ws_ref, wn_ref, b_ref, o_ref, *,
                   relu):
    msum = m_ref[0] + m_ref[1]
    deg = jnp.maximum(d_ref[0, :, 0] + d_ref[1, :, 0], 1.0)
    agg = lax.dot(msum, wn_ref[...], precision=lax.Precision.HIGHEST)
    slf = lax.dot(x_ref[...], ws_ref[...], precision=lax.Precision.HIGHEST)
    out = slf + agg / deg[:, None] + b_ref[...]
    o_ref[...] = jnp.maximum(out, 0.0) if relu else out


def _make_tc_layer(relu):
    return pl.pallas_call(
        functools.partial(_tc_layer_body, relu=relu),
        grid=(N // RB,),
        in_specs=[
            pl.BlockSpec((RB, D), lambda i: (i, 0)),
            pl.BlockSpec((NC, RB, D), lambda i: (0, i, 0)),
            pl.BlockSpec((NC, RB, DEGW), lambda i: (0, i, 0)),
            pl.BlockSpec((D, D), lambda i: (0, 0)),
            pl.BlockSpec((D, D), lambda i: (0, 0)),
            pl.BlockSpec((1, D), lambda i: (0, 0)),
        ],
        out_specs=pl.BlockSpec((RB, D), lambda i: (i, 0)),
        out_shape=jax.ShapeDtypeStruct((N, D), jnp.float32),
    )


_tc_layer1 = _make_tc_layer(relu=True)
_tc_layer2 = _make_tc_layer(relu=False)


def kernel(x, edge_index, W_self1, W_neigh1, b1, W_self2, W_neigh2, b2):
    src = edge_index[0]
    dst = edge_index[1]
    zrow = jnp.zeros((RPT, D), jnp.float32)
    ones_h = jnp.ones((C, D), jnp.float32)
    degp = _deg(dst, zrow, ones_h)
    degc = lax.slice(degp, (0, 0, 0), (NC, N, DEGW))
    zrow2, _ = lax.optimization_barrier((zrow, degp))
    msum1 = _agg(x, src, dst, zrow2)
    h1 = _tc_layer1(x, msum1, degc, W_self1, W_neigh1, b1.reshape(1, D))
    msum2 = _agg(h1, src, dst, zrow)
    return _tc_layer2(h1, msum2, degc, W_self2, W_neigh2, b2.reshape(1, D))


# speedup vs baseline: 8.1725x; 1.0742x over previous
import functools

import jax
import jax.numpy as jnp
from jax import lax
from jax.experimental import pallas as pl
from jax.experimental.pallas import tpu as pltpu
from jax.experimental.pallas import tpu_sc as plsc

N = 10000
E = 320000
D = 128
NC = 2
NS = 16
NW = NC * NS
EPW = E // NW
C = 128
FULL = EPW // C
TAIL = EPW - FULL * C
RPT = 624
REM = N - NS * RPT

_mesh = plsc.VectorSubcoreMesh(core_axis_name="c", subcore_axis_name="s")


def _agg_body(h, src, dst, zrow, msum_o,
              acc, sidx0, didx0, rows0, sidx1, didx1, rows1,
              sidx_t, didx_t, rows_t, sem0, sem1, semi0, semi1):
    _agg_pipeline(h, src, dst, zrow, msum_o, acc, sidx0, didx0, rows0,
                  sidx1, didx1, rows1, sidx_t, didx_t, rows_t,
                  sem0, sem1, semi0, semi1)


def _zero_acc(zrow, acc, sid):
    pltpu.sync_copy(zrow, acc.at[pl.ds(sid * RPT, RPT)])

    @pl.when(sid == 0)
    def _():
        pltpu.sync_copy(zrow.at[pl.ds(0, REM)], acc.at[pl.ds(NS * RPT, REM)])


def _drain_acc(acc, out, cid, sid):
    pltpu.sync_copy(acc.at[pl.ds(sid * RPT, RPT)],
                    out.at[cid, pl.ds(sid * RPT, RPT)])

    @pl.when(sid == 0)
    def _():
        pltpu.sync_copy(acc.at[pl.ds(NS * RPT, REM)],
                        out.at[cid, pl.ds(NS * RPT, REM)])


def _agg_pipeline(h, src, dst, zrow, msum_o,
                  acc, sidx0, didx0, rows0, sidx1, didx1, rows1,
                  sidx_t, didx_t, rows_t, sem0, sem1, semi0, semi1):
    cid = lax.axis_index("c")
    sid = lax.axis_index("s")
    wid = cid * NS + sid
    _zero_acc(zrow, acc, sid)
    plsc.subcore_barrier()
    base = wid * EPW
    _gather_scatter(h, src, dst, acc, sidx0, didx0, rows0, sidx1, didx1,
                    rows1, sem0, sem1, semi0, semi1, base, FULL)
    off = base + FULL * C
    pltpu.sync_copy(src.at[pl.ds(off, TAIL)], sidx_t)
    pltpu.sync_copy(dst.at[pl.ds(off, TAIL)], didx_t)
    pltpu.async_copy(h.at[sidx_t], rows_t, sem0).wait()
    pltpu.sync_copy(rows_t, acc.at[didx_t], add=True)
    plsc.subcore_barrier()
    _drain_acc(acc, msum_o, cid, sid)


def _gather_scatter(h, src, dst, acc, sidx0, didx0, rows0, sidx1, didx1,
                    rows1, sem0, sem1, semi0, semi1, base, n_full):
    def fire_idx(off, sidx, didx, semi):
        pltpu.async_copy(src.at[pl.ds(off, C)], sidx, semi)
        pltpu.async_copy(dst.at[pl.ds(off, C)], didx, semi)

    def wait_idx(sidx, didx, semi):
        pltpu.make_async_copy(src.at[pl.ds(base, C)], sidx, semi).wait()
        pltpu.make_async_copy(dst.at[pl.ds(base, C)], didx, semi).wait()

    fire_idx(base, sidx0, didx0, semi0)
    wait_idx(sidx0, didx0, semi0)
    pltpu.async_copy(h.at[sidx0], rows0, sem0)
    fire_idx(base + C, sidx1, didx1, semi1)
    LAST = n_full // 2 - 1

    @pl.loop(0, n_full // 2)
    def _(j):
        c0 = base + (2 * j) * C
        wait_idx(sidx1, didx1, semi1)
        pltpu.async_copy(h.at[sidx1], rows1, sem1)
        pltpu.make_async_copy(h.at[sidx0], rows0, sem0).wait()
        pltpu.sync_copy(rows0, acc.at[didx0], add=True)

        @pl.when(j < LAST)
        def _():
            fire_idx(c0 + 2 * C, sidx0, didx0, semi0)

        pltpu.make_async_copy(h.at[sidx1], rows1, sem1).wait()
        pltpu.sync_copy(rows1, acc.at[didx1], add=True)

        @pl.when(j < LAST)
        def _():
            fire_idx(c0 + 3 * C, sidx1, didx1, semi1)
            wait_idx(sidx0, didx0, semi0)
            pltpu.async_copy(h.at[sidx0], rows0, sem0)


def _make_agg(W):
    return pl.kernel(
        _agg_body,
        out_type=jax.ShapeDtypeStruct((NC, N, W), jnp.float32),
        mesh=_mesh,
        scratch_types=[
            pltpu.VMEM_SHARED((N, W), jnp.float32),
            pltpu.VMEM((C,), jnp.int32),
            pltpu.VMEM((C,), jnp.int32),
            pltpu.VMEM((C, W), jnp.float32),
            pltpu.VMEM((C,), jnp.int32),
            pltpu.VMEM((C,), jnp.int32),
            pltpu.VMEM((C, W), jnp.float32),
            pltpu.VMEM((TAIL,), jnp.int32),
            pltpu.VMEM((TAIL,), jnp.int32),
            pltpu.VMEM((TAIL, W), jnp.float32),
            pltpu.SemaphoreType.DMA,
            pltpu.SemaphoreType.DMA,
            pltpu.SemaphoreType.DMA,
            pltpu.SemaphoreType.DMA,
        ],
    )


_agg = _make_agg(D)


def _deg_body(dst, zrow, ones_h, deg_o,
              acc, didx0, didx1, ones_v, didx_t, semi0, semi1):
    cid = lax.axis_index("c")
    sid = lax.axis_index("s")
    wid = cid * NS + sid
    _zero_acc(zrow, acc, sid)
    pltpu.sync_copy(ones_h, ones_v)
    plsc.subcore_barrier()
    base = wid * EPW
    pltpu.async_copy(dst.at[pl.ds(base, C)], didx0, semi0)
    LAST = FULL // 2 - 1

    @pl.loop(0, FULL // 2)
    def _(j):
        c0 = base + (2 * j) * C
        pltpu.async_copy(dst.at[pl.ds(c0 + C, C)], didx1, semi1)
        pltpu.make_async_copy(dst.at[pl.ds(base, C)], didx0, semi0).wait()
        pltpu.sync_copy(ones_v, acc.at[didx0], add=True)

        @pl.when(j < LAST)
        def _():
            pltpu.async_copy(dst.at[pl.ds(c0 + 2 * C, C)], didx0, semi0)

        pltpu.make_async_copy(dst.at[pl.ds(base, C)], didx1, semi1).wait()
        pltpu.sync_copy(ones_v, acc.at[didx1], add=True)

    pltpu.sync_copy(dst.at[pl.ds(base + FULL * C, TAIL)], didx_t)
    pltpu.sync_copy(ones_v.at[pl.ds(0, TAIL)], acc.at[didx_t], add=True)

    plsc.subcore_barrier()
    _drain_acc(acc, deg_o, cid, sid)


_deg = pl.kernel(
    _deg_body,
    out_type=jax.ShapeDtypeStruct((NC, N, D), jnp.float32),
    mesh=_mesh,
    scratch_types=[
        pltpu.VMEM_SHARED((N, D), jnp.float32),
        pltpu.VMEM((C,), jnp.int32),
        pltpu.VMEM((C,), jnp.int32),
        pltpu.VMEM((C, D), jnp.float32),
        pltpu.VMEM((TAIL,), jnp.int32),
        pltpu.SemaphoreType.DMA,
        pltpu.SemaphoreType.DMA,
    ],
)

RB = 1000
DEGW = 16


def _tc_layer_body(x_ref, m_ref, d_ref, ws_ref, wn_ref, b_ref, o_ref, *,
                   relu):
    msum = m_ref[0] + m_ref[1]
    deg = jnp.maximum(d_ref[0, :, 0] + d_ref[1, :, 0], 1.0)
    agg = lax.dot(msum, wn_ref[...], precision=lax.Precision.HIGHEST)
    slf = lax.dot(x_ref[...], ws_ref[...], precision=lax.Precision.HIGHEST)
    out = slf + agg / deg[:, None] + b_ref[...]
    o_ref[...] = jnp.maximum(out, 0.0) if relu else out


def _make_tc_layer(relu):
    return pl.pallas_call(
        functools.partial(_tc_layer_body, relu=relu),
        grid=(N // RB,),
        in_specs=[
            pl.BlockSpec((RB, D), lambda i: (i, 0)),
            pl.BlockSpec((NC, RB, D), lambda i: (0, i, 0)),
            pl.BlockSpec((NC, RB, DEGW), lambda i: (0, i, 0)),
            pl.BlockSpec((D, D), lambda i: (0, 0)),
            pl.BlockSpec((D, D), lambda i: (0, 0)),
            pl.BlockSpec((1, D), lambda i: (0, 0)),
        ],
        out_specs=pl.BlockSpec((RB, D), lambda i: (i, 0)),
        out_shape=jax.ShapeDtypeStruct((N, D), jnp.float32),
    )


_tc_layer1 = _make_tc_layer(relu=True)
_tc_layer2 = _make_tc_layer(relu=False)


def kernel(x, edge_index, W_self1, W_neigh1, b1, W_self2, W_neigh2, b2):
    src = edge_index[0]
    dst = edge_index[1]
    zrow = jnp.zeros((RPT, D), jnp.float32)
    ones_h = jnp.ones((C, D), jnp.float32)
    degp = _deg(dst, zrow, ones_h)
    degc = lax.slice(degp, (0, 0, 0), (NC, N, DEGW))
    zrow2, _ = lax.optimization_barrier((zrow, degp))
    msum1 = _agg(x, src, dst, zrow2)
    h1 = _tc_layer1(x, msum1, degc, W_self1, W_neigh1, b1.reshape(1, D))
    msum2 = _agg(h1, src, dst, zrow)
    return _tc_layer2(h1, msum2, degc, W_self2, W_neigh2, b2.reshape(1, D))
